# baseline (speedup 1.0000x reference)
import functools

import jax
import jax.numpy as jnp
from jax import lax
from jax.experimental import pallas as pl
from jax.experimental.pallas import tpu as pltpu
from jax.experimental.pallas import tpu_sc as plsc

_EDGE_TYPES = [
    ("bus", "ac_line", "bus"),
    ("bus", "transformer", "bus"),
    ("generator", "generator_link", "bus"),
    ("bus", "generator_link", "generator"),
    ("load", "load_link", "bus"),
    ("bus", "load_link", "load"),
    ("shunt", "shunt_link", "bus"),
    ("bus", "shunt_link", "shunt"),
]
_N_NODES = {"bus": 50000, "generator": 10000, "load": 25000, "shunt": 5000}
_IN_DIMS = {"bus": 32, "generator": 16, "load": 16, "shunt": 8}
_HIDDEN = 64
_NODE_TYPES = ["bus", "generator", "load", "shunt"]

_EDGE_CFG = {
    "bus_ac_line_bus": (800000, 3, 132),
    "bus_transformer_bus": (100000, 2, 26),
    "generator_generator_link_bus": (10000, 1, 6),
    "bus_generator_link_generator": (10000, 1, 6),
    "load_load_link_bus": (25000, 1, 14),
    "bus_load_link_load": (25000, 1, 14),
    "shunt_shunt_link_bus": (5000, 1, 4),
    "bus_shunt_link_shunt": (5000, 1, 4),
}
_N_ACC = {"bus": 50048, "generator": 10112, "load": 25088, "shunt": 5120}
_ZROWS = _N_ACC["bus"] // 16


def _ek(et):
    return et[0] + "_" + et[1] + "_" + et[2]


@functools.lru_cache(maxsize=None)
def _seg_sum_kernel(n_acc, e_pad, k_batches, nch):
    es = e_pad // 16
    rs = es // 128
    rpt = n_acc // 16
    mesh = plsc.VectorSubcoreMesh(core_axis_name="c", subcore_axis_name="s")

    @functools.partial(
        pl.kernel,
        out_type=(jax.ShapeDtypeStruct((n_acc, 32), jnp.float32),
                  jax.ShapeDtypeStruct((n_acc, 32), jnp.float32)),
        mesh=mesh,
        scratch_types=[
            pltpu.VMEM_SHARED((n_acc, 32), jnp.float32),
            pltpu.VMEM((2, 2, k_batches, 128), jnp.int32),
            pltpu.VMEM((2, k_batches, 128, 32), jnp.float32),
            pltpu.SemaphoreType.DMA,
        ],
        compiler_params=pltpu.CompilerParams(use_tc_tiling_on_sc=False),
    )
    def k(tab0, tab1, src2d, dst2d, zeros, out0, out1, acc, idxb, rows, sem):
        cid = lax.axis_index("c")
        sid = lax.axis_index("s")
        r0 = sid * rpt
        pltpu.sync_copy(zeros.at[pl.ds(0, rpt)], acc.at[pl.ds(r0, rpt)])
        plsc.subcore_barrier()

        def load_fire(j, b):
            rb = sid * rs + j * k_batches
            pltpu.sync_copy(src2d.at[pl.ds(rb, k_batches)], idxb.at[b, 0])
            pltpu.sync_copy(dst2d.at[pl.ds(rb, k_batches)], idxb.at[b, 1])

            @pl.when(cid == 0)
            def _():
                for t in range(k_batches):
                    pltpu.async_copy(tab0.at[idxb.at[b, 0, t]],
                                     rows.at[b, t], sem)

            @pl.when(cid == 1)
            def _():
                for t in range(k_batches):
                    pltpu.async_copy(tab1.at[idxb.at[b, 0, t]],
                                     rows.at[b, t], sem)

        def drain_scatter(b):
            @pl.when(cid == 0)
            def _():
                for t in range(k_batches):
                    pltpu.make_async_copy(tab0.at[idxb.at[b, 0, t]],
                                          rows.at[b, t], sem).wait()

            @pl.when(cid == 1)
            def _():
                for t in range(k_batches):
                    pltpu.make_async_copy(tab1.at[idxb.at[b, 0, t]],
                                          rows.at[b, t], sem).wait()

            for t in range(k_batches):
                pltpu.sync_copy(rows.at[b, t], acc.at[idxb.at[b, 1, t]],
                                add=True)

        npairs = nch // 2
        load_fire(0, 0)

        def pair(p, carry):
            load_fire(2 * p + 1, 1)
            drain_scatter(0)

            @pl.when(p < npairs - 1)
            def _():
                load_fire(2 * p + 2, 0)

            drain_scatter(1)
            return carry

        lax.fori_loop(0, npairs, pair, 0)
        plsc.subcore_barrier()

        @pl.when(cid == 0)
        def _():
            pltpu.sync_copy(acc.at[pl.ds(r0, rpt)], out0.at[pl.ds(r0, rpt)])

        @pl.when(cid == 1)
        def _():
            pltpu.sync_copy(acc.at[pl.ds(r0, rpt)], out1.at[pl.ds(r0, rpt)])

    return k


def _seg_sum(ekey, tab0, tab1, src2d, dst2d, zeros):
    e, kb, nch = _EDGE_CFG[ekey]
    n_acc = _N_ACC[ekey.split("_")[-1]]
    e_pad = 16 * 128 * kb * nch
    return _seg_sum_kernel(n_acc, e_pad, kb, nch)(tab0, tab1, src2d, dst2d, zeros)


def _inv_counts(cnt_slab):
    n_acc = cnt_slab.shape[0]

    def body(c_ref, o_ref):
        o_ref[...] = 1.0 / jnp.maximum(c_ref[...][:, 0:1], 1.0)

    return pl.pallas_call(
        body,
        grid=(pl.cdiv(n_acc, 2048),),
        in_specs=[pl.BlockSpec((2048, 32), lambda i: (i, 0))],
        out_specs=pl.BlockSpec((2048, 1), lambda i: (i, 0)),
        out_shape=jax.ShapeDtypeStruct((n_acc, 1), jnp.float32),
    )(cnt_slab)


def _combine(n_rows, cnts, contribs, x_terms, bias, head, block=1024):
    grid = (pl.cdiv(n_rows, block),)
    n_cnt, n_s, n_x = len(cnts), len(contribs), len(x_terms)

    def body(*refs):
        i = 0
        cnt_refs = refs[i:i + n_cnt]; i += n_cnt
        s_refs = refs[i:i + n_s]; i += n_s
        wl_refs = refs[i:i + n_s]; i += n_s
        x_refs = refs[i:i + n_x]; i += n_x
        wx_refs = refs[i:i + n_x]; i += n_x
        b_ref = refs[i]; i += 1
        if head is not None:
            w1_ref, b1_ref, w2_ref, b2_ref = refs[i:i + 4]; i += 4
        out_refs = refs[i:]

        acc = jnp.broadcast_to(b_ref[0], (block, _HIDDEN))
        for xr, wr in zip(x_refs, wx_refs):
            acc = acc + jnp.dot(xr[...], wr[...],
                                preferred_element_type=jnp.float32)
        inv = [cr[...] for cr in cnt_refs]
        for (ci, _, _), sr, wl in zip(contribs, s_refs, wl_refs):
            acc = acc + jnp.dot(sr[...] * inv[ci], wl[...],
                                preferred_element_type=jnp.float32)
        h = jnp.maximum(acc, 0.0)
        if head is None:
            out_refs[0][...] = h[:, :32]
            out_refs[1][...] = h[:, 32:]
        else:
            u = jnp.maximum(jnp.dot(h, w1_ref[...],
                                    preferred_element_type=jnp.float32)
                            + b1_ref[0], 0.0)
            out_refs[0][...] = (jnp.dot(u, w2_ref[...],
                                        preferred_element_type=jnp.float32)
                               + b2_ref[0])

    def row_spec(w):
        return pl.BlockSpec((block, w), lambda i: (i, 0))

    def full_spec(a):
        nd = a.ndim
        return pl.BlockSpec(a.shape, lambda i, _nd=nd: (0,) * _nd)

    args = []
    specs = []
    for c in cnts:
        args.append(c); specs.append(row_spec(1))
    for _, s, _ in contribs:
        args.append(s); specs.append(row_spec(32))
    for _, _, wl in contribs:
        args.append(wl); specs.append(full_spec(wl))
    for x, _ in x_terms:
        args.append(x); specs.append(row_spec(x.shape[1]))
    for _, w in x_terms:
        args.append(w); specs.append(full_spec(w))
    b2d = bias.reshape(1, _HIDDEN)
    args.append(b2d); specs.append(full_spec(b2d))
    if head is not None:
        w1, b1, w2, b2 = head
        for a in (w1, b1.reshape(1, -1), w2, b2.reshape(1, -1)):
            args.append(a); specs.append(full_spec(a))
        od = head[2].shape[1]
        out_shape = [jax.ShapeDtypeStruct((n_rows, od), jnp.float32)]
        out_specs = [pl.BlockSpec((block, od), lambda i: (i, 0))]
    else:
        out_shape = [jax.ShapeDtypeStruct((n_rows, 32), jnp.float32)] * 2
        out_specs = [pl.BlockSpec((block, 32), lambda i: (i, 0))] * 2

    res = pl.pallas_call(
        body,
        grid=grid,
        in_specs=specs,
        out_specs=out_specs,
        out_shape=out_shape,
    )(*args)
    return res[0] if head is not None else tuple(res)


def kernel(x_bus, x_generator, x_load, x_shunt,
           edge_index_bus_ac_line_bus, edge_index_bus_transformer_bus,
           edge_index_generator_generator_link_bus,
           edge_index_bus_generator_link_generator,
           edge_index_load_load_link_bus, edge_index_bus_load_link_load,
           edge_index_shunt_shunt_link_bus, edge_index_bus_shunt_link_shunt,
           params):
    x0 = {"bus": x_bus, "generator": x_generator,
          "load": x_load, "shunt": x_shunt}
    ei = {
        "bus_ac_line_bus": edge_index_bus_ac_line_bus,
        "bus_transformer_bus": edge_index_bus_transformer_bus,
        "generator_generator_link_bus": edge_index_generator_generator_link_bus,
        "bus_generator_link_generator": edge_index_bus_generator_link_generator,
        "load_load_link_bus": edge_index_load_load_link_bus,
        "bus_load_link_load": edge_index_bus_load_link_load,
        "shunt_shunt_link_bus": edge_index_shunt_shunt_link_bus,
        "bus_shunt_link_shunt": edge_index_bus_shunt_link_shunt,
    }

    zeros = jnp.zeros((_ZROWS, 32), jnp.float32)

    src2d, dst2d = {}, {}
    for et in _EDGE_TYPES:
        ekey = _ek(et)
        e, kb, nch = _EDGE_CFG[ekey]
        e_pad = 16 * 128 * kb * nch
        s = ei[ekey][0].astype(jnp.int32)
        d = ei[ekey][1].astype(jnp.int32)
        pad = e_pad - e
        s = jnp.concatenate([s, jnp.zeros((pad,), jnp.int32)])
        d = jnp.concatenate([d, jnp.full((pad,), _N_NODES[et[2]], jnp.int32)])
        src2d[ekey] = s.reshape(e_pad // 128, 128)
        dst2d[ekey] = d.reshape(e_pad // 128, 128)

    tab_pad = {nt: jnp.pad(x0[nt], ((0, 0), (0, 32 - _IN_DIMS[nt])))
               for nt in _NODE_TYPES}
    tab_one = {nt: jnp.ones((_N_NODES[nt], 32), jnp.float32)
               for nt in _NODE_TYPES}

    edge_types_of = lambda layer: (
        [et for et in _EDGE_TYPES if et[2] in ("bus", "generator")]
        if layer == 2 else _EDGE_TYPES)
    dst_types_of = lambda layer: (
        ["bus", "generator"] if layer == 2 else _NODE_TYPES)

    cnt = {}
    h_half = None
    out_bus = out_gen = None

    for layer in range(3):
        segsums = {}
        for et in edge_types_of(layer):
            ekey = _ek(et)
            if layer == 0:
                res = _seg_sum(ekey, tab_pad[et[0]], tab_one[et[0]],
                               src2d[ekey], dst2d[ekey], zeros)
                cnt[ekey] = _inv_counts(res[1])
                wl = params["l0_%s_Wl" % ekey]
                wl = jnp.pad(wl, ((0, 32 - wl.shape[0]), (0, 0)))
                segsums[ekey] = ([res[0]], [wl])
            else:
                lo, hi = h_half[et[0]]
                res = _seg_sum(ekey, lo, hi, src2d[ekey], dst2d[ekey], zeros)
                wl = params["l%d_%s_Wl" % (layer, ekey)]
                segsums[ekey] = ([res[0], res[1]], [wl[:32], wl[32:]])

        new_h = {}
        for nt in dst_types_of(layer):
            in_ets = [et for et in edge_types_of(layer) if et[2] == nt]
            cnts, contribs = [], []
            bias = 0.0
            x_terms = []
            for et in in_ets:
                ekey = _ek(et)
                ci = len(cnts)
                cnts.append(cnt[ekey])
                s_arrays, wl_parts = segsums[ekey]
                for s, wl in zip(s_arrays, wl_parts):
                    contribs.append((ci, s, wl))
                wr = params["l%d_%s_Wr" % (layer, ekey)]
                if layer == 0:
                    x_terms.append((x0[nt], wr))
                else:
                    lo, hi = h_half[nt]
                    x_terms.append((lo, wr[:32]))
                    x_terms.append((hi, wr[32:]))
                bias = bias + params["l%d_%s_bl" % (layer, ekey)]
            head = None
            if layer == 2:
                head = (params[nt + "_W1"], params[nt + "_b1"],
                        params[nt + "_W2"], params[nt + "_b2"])
            res = _combine(_N_NODES[nt], cnts, contribs, x_terms, bias, head)
            if layer == 2:
                if nt == "bus":
                    out_bus = res
                else:
                    out_gen = res
            else:
                new_h[nt] = res
        if layer < 2:
            h_half = new_h

    return out_bus, out_gen


# speedup vs baseline: 1.0421x; 1.0421x over previous
import functools

import jax
import jax.numpy as jnp
from jax import lax
from jax.experimental import pallas as pl
from jax.experimental.pallas import tpu as pltpu
from jax.experimental.pallas import tpu_sc as plsc

_EDGE_TYPES = [
    ("bus", "ac_line", "bus"),
    ("bus", "transformer", "bus"),
    ("generator", "generator_link", "bus"),
    ("bus", "generator_link", "generator"),
    ("load", "load_link", "bus"),
    ("bus", "load_link", "load"),
    ("shunt", "shunt_link", "bus"),
    ("bus", "shunt_link", "shunt"),
]
_N_NODES = {"bus": 50000, "generator": 10000, "load": 25000, "shunt": 5000}
_IN_DIMS = {"bus": 32, "generator": 16, "load": 16, "shunt": 8}
_HIDDEN = 64
_NODE_TYPES = ["bus", "generator", "load", "shunt"]

_EDGE_CFG = {
    "bus_ac_line_bus": (800000, 2, 196),
    "bus_transformer_bus": (100000, 2, 26),
    "generator_generator_link_bus": (10000, 1, 6),
    "bus_generator_link_generator": (10000, 1, 6),
    "load_load_link_bus": (25000, 1, 14),
    "bus_load_link_load": (25000, 1, 14),
    "shunt_shunt_link_bus": (5000, 1, 4),
    "bus_shunt_link_shunt": (5000, 1, 4),
}
_N_ACC = {"bus": 50048, "generator": 10112, "load": 25088, "shunt": 5120}
_ZROWS = _N_ACC["bus"] // 16


def _ek(et):
    return et[0] + "_" + et[1] + "_" + et[2]


@functools.lru_cache(maxsize=None)
def _seg_sum_kernel(n_acc, e_pad, k_batches, nch):
    es = e_pad // 16
    rs = es // 128
    rpt = n_acc // 16
    mesh = plsc.VectorSubcoreMesh(core_axis_name="c", subcore_axis_name="s")

    @functools.partial(
        pl.kernel,
        out_type=(jax.ShapeDtypeStruct((n_acc, 32), jnp.float32),
                  jax.ShapeDtypeStruct((n_acc, 32), jnp.float32)),
        mesh=mesh,
        scratch_types=[
            pltpu.VMEM_SHARED((n_acc, 32), jnp.float32),
            pltpu.VMEM((2, 2, k_batches, 128), jnp.int32),
            pltpu.VMEM((2, k_batches, 128, 32), jnp.float32),
            pltpu.SemaphoreType.DMA,
        ],
        compiler_params=pltpu.CompilerParams(use_tc_tiling_on_sc=False),
    )
    def k(tab0, tab1, src2d, dst2d, zeros, out0, out1, acc, idxb, rows, sem):
        cid = lax.axis_index("c")
        sid = lax.axis_index("s")
        r0 = sid * rpt
        pltpu.sync_copy(zeros.at[pl.ds(0, rpt)], acc.at[pl.ds(r0, rpt)])
        plsc.subcore_barrier()

        def load_fire(j, b):
            rb = sid * rs + j * k_batches
            pltpu.sync_copy(src2d.at[pl.ds(rb, k_batches)], idxb.at[b, 0])
            pltpu.sync_copy(dst2d.at[pl.ds(rb, k_batches)], idxb.at[b, 1])

            @pl.when(cid == 0)
            def _():
                for t in range(k_batches):
                    pltpu.async_copy(tab0.at[idxb.at[b, 0, t]],
                                     rows.at[b, t], sem)

            @pl.when(cid == 1)
            def _():
                for t in range(k_batches):
                    pltpu.async_copy(tab1.at[idxb.at[b, 0, t]],
                                     rows.at[b, t], sem)

        def drain_scatter(b):
            @pl.when(cid == 0)
            def _():
                for t in range(k_batches):
                    pltpu.make_async_copy(tab0.at[idxb.at[b, 0, t]],
                                          rows.at[b, t], sem).wait()

            @pl.when(cid == 1)
            def _():
                for t in range(k_batches):
                    pltpu.make_async_copy(tab1.at[idxb.at[b, 0, t]],
                                          rows.at[b, t], sem).wait()

            for t in range(k_batches):
                pltpu.sync_copy(rows.at[b, t], acc.at[idxb.at[b, 1, t]],
                                add=True)

        npairs = nch // 2
        load_fire(0, 0)

        def pair(p, carry):
            load_fire(2 * p + 1, 1)
            drain_scatter(0)

            @pl.when(p < npairs - 1)
            def _():
                load_fire(2 * p + 2, 0)

            drain_scatter(1)
            return carry

        lax.fori_loop(0, npairs, pair, 0)
        plsc.subcore_barrier()

        @pl.when(cid == 0)
        def _():
            pltpu.sync_copy(acc.at[pl.ds(r0, rpt)], out0.at[pl.ds(r0, rpt)])

        @pl.when(cid == 1)
        def _():
            pltpu.sync_copy(acc.at[pl.ds(r0, rpt)], out1.at[pl.ds(r0, rpt)])

    return k


def _seg_sum(ekey, tab0, tab1, src2d, dst2d, zeros):
    e, kb, nch = _EDGE_CFG[ekey]
    n_acc = _N_ACC[ekey.split("_")[-1]]
    e_pad = 16 * 128 * kb * nch
    return _seg_sum_kernel(n_acc, e_pad, kb, nch)(tab0, tab1, src2d, dst2d, zeros)


def _inv_counts(cnt_slab):
    n_acc = cnt_slab.shape[0]

    def body(c_ref, o_ref):
        o_ref[...] = 1.0 / jnp.maximum(c_ref[...][:, 0:1], 1.0)

    return pl.pallas_call(
        body,
        grid=(pl.cdiv(n_acc, 2048),),
        in_specs=[pl.BlockSpec((2048, 32), lambda i: (i, 0))],
        out_specs=pl.BlockSpec((2048, 1), lambda i: (i, 0)),
        out_shape=jax.ShapeDtypeStruct((n_acc, 1), jnp.float32),
    )(cnt_slab)


def _combine(n_rows, cnts, contribs, x_terms, bias, head, block=1024):
    grid = (pl.cdiv(n_rows, block),)
    n_cnt, n_s, n_x = len(cnts), len(contribs), len(x_terms)

    def body(*refs):
        i = 0
        cnt_refs = refs[i:i + n_cnt]; i += n_cnt
        s_refs = refs[i:i + n_s]; i += n_s
        wl_refs = refs[i:i + n_s]; i += n_s
        x_refs = refs[i:i + n_x]; i += n_x
        wx_refs = refs[i:i + n_x]; i += n_x
        b_ref = refs[i]; i += 1
        if head is not None:
            w1_ref, b1_ref, w2_ref, b2_ref = refs[i:i + 4]; i += 4
        out_refs = refs[i:]

        acc = jnp.broadcast_to(b_ref[0], (block, _HIDDEN))
        for xr, wr in zip(x_refs, wx_refs):
            acc = acc + jnp.dot(xr[...], wr[...],
                                preferred_element_type=jnp.float32)
        inv = [cr[...] for cr in cnt_refs]
        for (ci, _, _), sr, wl in zip(contribs, s_refs, wl_refs):
            acc = acc + jnp.dot(sr[...] * inv[ci], wl[...],
                                preferred_element_type=jnp.float32)
        h = jnp.maximum(acc, 0.0)
        if head is None:
            out_refs[0][...] = h[:, :32]
            out_refs[1][...] = h[:, 32:]
        else:
            u = jnp.maximum(jnp.dot(h, w1_ref[...],
                                    preferred_element_type=jnp.float32)
                            + b1_ref[0], 0.0)
            out_refs[0][...] = (jnp.dot(u, w2_ref[...],
                                        preferred_element_type=jnp.float32)
                               + b2_ref[0])

    def row_spec(w):
        return pl.BlockSpec((block, w), lambda i: (i, 0))

    def full_spec(a):
        nd = a.ndim
        return pl.BlockSpec(a.shape, lambda i, _nd=nd: (0,) * _nd)

    args = []
    specs = []
    for c in cnts:
        args.append(c); specs.append(row_spec(1))
    for _, s, _ in contribs:
        args.append(s); specs.append(row_spec(32))
    for _, _, wl in contribs:
        args.append(wl); specs.append(full_spec(wl))
    for x, _ in x_terms:
        args.append(x); specs.append(row_spec(x.shape[1]))
    for _, w in x_terms:
        args.append(w); specs.append(full_spec(w))
    b2d = bias.reshape(1, _HIDDEN)
    args.append(b2d); specs.append(full_spec(b2d))
    if head is not None:
        w1, b1, w2, b2 = head
        for a in (w1, b1.reshape(1, -1), w2, b2.reshape(1, -1)):
            args.append(a); specs.append(full_spec(a))
        od = head[2].shape[1]
        out_shape = [jax.ShapeDtypeStruct((n_rows, od), jnp.float32)]
        out_specs = [pl.BlockSpec((block, od), lambda i: (i, 0))]
    else:
        out_shape = [jax.ShapeDtypeStruct((n_rows, 32), jnp.float32)] * 2
        out_specs = [pl.BlockSpec((block, 32), lambda i: (i, 0))] * 2

    res = pl.pallas_call(
        body,
        grid=grid,
        in_specs=specs,
        out_specs=out_specs,
        out_shape=out_shape,
    )(*args)
    return res[0] if head is not None else tuple(res)


def kernel(x_bus, x_generator, x_load, x_shunt,
           edge_index_bus_ac_line_bus, edge_index_bus_transformer_bus,
           edge_index_generator_generator_link_bus,
           edge_index_bus_generator_link_generator,
           edge_index_load_load_link_bus, edge_index_bus_load_link_load,
           edge_index_shunt_shunt_link_bus, edge_index_bus_shunt_link_shunt,
           params):
    x0 = {"bus": x_bus, "generator": x_generator,
          "load": x_load, "shunt": x_shunt}
    ei = {
        "bus_ac_line_bus": edge_index_bus_ac_line_bus,
        "bus_transformer_bus": edge_index_bus_transformer_bus,
        "generator_generator_link_bus": edge_index_generator_generator_link_bus,
        "bus_generator_link_generator": edge_index_bus_generator_link_generator,
        "load_load_link_bus": edge_index_load_load_link_bus,
        "bus_load_link_load": edge_index_bus_load_link_load,
        "shunt_shunt_link_bus": edge_index_shunt_shunt_link_bus,
        "bus_shunt_link_shunt": edge_index_bus_shunt_link_shunt,
    }

    zeros = jnp.zeros((_ZROWS, 32), jnp.float32)

    src2d, dst2d = {}, {}
    for et in _EDGE_TYPES:
        ekey = _ek(et)
        e, kb, nch = _EDGE_CFG[ekey]
        e_pad = 16 * 128 * kb * nch
        s = ei[ekey][0].astype(jnp.int32)
        d = ei[ekey][1].astype(jnp.int32)
        pad = e_pad - e
        s = jnp.concatenate([s, jnp.zeros((pad,), jnp.int32)])
        d = jnp.concatenate([d, jnp.full((pad,), _N_NODES[et[2]], jnp.int32)])
        src2d[ekey] = s.reshape(e_pad // 128, 128)
        dst2d[ekey] = d.reshape(e_pad // 128, 128)

    tab_pad = {nt: jnp.pad(x0[nt], ((0, 0), (0, 32 - _IN_DIMS[nt])))
               for nt in _NODE_TYPES}
    tab_one = {nt: jnp.ones((_N_NODES[nt], 32), jnp.float32)
               for nt in _NODE_TYPES}

    edge_types_of = lambda layer: (
        [et for et in _EDGE_TYPES if et[2] in ("bus", "generator")]
        if layer == 2 else _EDGE_TYPES)
    dst_types_of = lambda layer: (
        ["bus", "generator"] if layer == 2 else _NODE_TYPES)

    cnt = {}
    h_half = None
    out_bus = out_gen = None

    for layer in range(3):
        segsums = {}
        for et in edge_types_of(layer):
            ekey = _ek(et)
            if layer == 0:
                res = _seg_sum(ekey, tab_pad[et[0]], tab_one[et[0]],
                               src2d[ekey], dst2d[ekey], zeros)
                cnt[ekey] = _inv_counts(res[1])
                wl = params["l0_%s_Wl" % ekey]
                wl = jnp.pad(wl, ((0, 32 - wl.shape[0]), (0, 0)))
                segsums[ekey] = ([res[0]], [wl])
            else:
                lo, hi = h_half[et[0]]
                res = _seg_sum(ekey, lo, hi, src2d[ekey], dst2d[ekey], zeros)
                wl = params["l%d_%s_Wl" % (layer, ekey)]
                segsums[ekey] = ([res[0], res[1]], [wl[:32], wl[32:]])

        new_h = {}
        for nt in dst_types_of(layer):
            in_ets = [et for et in edge_types_of(layer) if et[2] == nt]
            cnts, contribs = [], []
            bias = 0.0
            x_terms = []
            for et in in_ets:
                ekey = _ek(et)
                ci = len(cnts)
                cnts.append(cnt[ekey])
                s_arrays, wl_parts = segsums[ekey]
                for s, wl in zip(s_arrays, wl_parts):
                    contribs.append((ci, s, wl))
                wr = params["l%d_%s_Wr" % (layer, ekey)]
                if layer == 0:
                    x_terms.append((x0[nt], wr))
                else:
                    lo, hi = h_half[nt]
                    x_terms.append((lo, wr[:32]))
                    x_terms.append((hi, wr[32:]))
                bias = bias + params["l%d_%s_bl" % (layer, ekey)]
            head = None
            if layer == 2:
                head = (params[nt + "_W1"], params[nt + "_b1"],
                        params[nt + "_W2"], params[nt + "_b2"])
            res = _combine(_N_NODES[nt], cnts, contribs, x_terms, bias, head)
            if layer == 2:
                if nt == "bus":
                    out_bus = res
                else:
                    out_gen = res
            else:
                new_h[nt] = res
        if layer < 2:
            h_half = new_h

    return out_bus, out_gen


# speedup vs baseline: 1.0698x; 1.0266x over previous
import functools

import jax
import jax.numpy as jnp
from jax import lax
from jax.experimental import pallas as pl
from jax.experimental.pallas import tpu as pltpu
from jax.experimental.pallas import tpu_sc as plsc

_EDGE_TYPES = [
    ("bus", "ac_line", "bus"),
    ("bus", "transformer", "bus"),
    ("generator", "generator_link", "bus"),
    ("bus", "generator_link", "generator"),
    ("load", "load_link", "bus"),
    ("bus", "load_link", "load"),
    ("shunt", "shunt_link", "bus"),
    ("bus", "shunt_link", "shunt"),
]
_N_NODES = {"bus": 50000, "generator": 10000, "load": 25000, "shunt": 5000}
_IN_DIMS = {"bus": 32, "generator": 16, "load": 16, "shunt": 8}
_HIDDEN = 64
_NODE_TYPES = ["bus", "generator", "load", "shunt"]

_EDGE_CFG = {
    "bus_ac_line_bus": (800000, 2, 196),
    "bus_transformer_bus": (100000, 2, 26),
    "generator_generator_link_bus": (10000, 1, 6),
    "bus_generator_link_generator": (10000, 1, 6),
    "load_load_link_bus": (25000, 1, 14),
    "bus_load_link_load": (25000, 1, 14),
    "shunt_shunt_link_bus": (5000, 1, 4),
    "bus_shunt_link_shunt": (5000, 1, 4),
}
_N_ACC = {"bus": 50048, "generator": 10112, "load": 25088, "shunt": 5120}
_ZROWS = _N_ACC["bus"] // 16
_SLAB_BASE = {"bus_generator_link_generator": 0,
              "bus_load_link_load": 10112,
              "bus_shunt_link_shunt": 35200}


def _ek(et):
    return et[0] + "_" + et[1] + "_" + et[2]


@functools.lru_cache(maxsize=None)
def _multi_seg_sum_kernel(jobs):
    kmax = max(j[3] for j in jobs)
    slab = max(j[0] + j[1] for j in jobs)
    mesh = plsc.VectorSubcoreMesh(core_axis_name="c", subcore_axis_name="s")
    out_type = []
    for n_acc, _, _, _, _ in jobs:
        out_type.append(jax.ShapeDtypeStruct((n_acc, 32), jnp.float32))
        out_type.append(jax.ShapeDtypeStruct((n_acc, 32), jnp.float32))

    @functools.partial(
        pl.kernel,
        out_type=tuple(out_type),
        mesh=mesh,
        scratch_types=[
            pltpu.VMEM_SHARED((slab, 32), jnp.float32),
            pltpu.VMEM((2, 2, kmax, 128), jnp.int32),
            pltpu.VMEM((2, kmax, 128, 32), jnp.float32),
            pltpu.SemaphoreType.DMA,
        ],
        compiler_params=pltpu.CompilerParams(use_tc_tiling_on_sc=False),
    )
    def k(*refs):
        n = len(jobs)
        ins = refs[:4 * n + 1]
        outs = refs[4 * n + 1:4 * n + 1 + 2 * n]
        acc, idxb, rows, sem = refs[4 * n + 1 + 2 * n:]
        zeros = ins[4 * n]
        cid = lax.axis_index("c")
        sid = lax.axis_index("s")

        for ji, (n_acc, base, e_pad, k_batches, nch) in enumerate(jobs):
            tab0, tab1, src2d, dst2d = ins[4 * ji:4 * ji + 4]
            out0, out1 = outs[2 * ji], outs[2 * ji + 1]
            rs = (e_pad // 16) // 128
            rpt = n_acc // 16
            r0 = sid * rpt
            pltpu.sync_copy(zeros.at[pl.ds(0, rpt)],
                            acc.at[pl.ds(base + r0, rpt)])
            plsc.subcore_barrier()

            def load_fire(j, b):
                rb = sid * rs + j * k_batches
                pltpu.sync_copy(src2d.at[pl.ds(rb, k_batches)], idxb.at[b, 0])
                pltpu.sync_copy(dst2d.at[pl.ds(rb, k_batches)], idxb.at[b, 1])

                @pl.when(cid == 0)
                def _():
                    for t in range(k_batches):
                        pltpu.async_copy(tab0.at[idxb.at[b, 0, t]],
                                         rows.at[b, t], sem)

                @pl.when(cid == 1)
                def _():
                    for t in range(k_batches):
                        pltpu.async_copy(tab1.at[idxb.at[b, 0, t]],
                                         rows.at[b, t], sem)

            def drain_scatter(b):
                @pl.when(cid == 0)
                def _():
                    for t in range(k_batches):
                        pltpu.make_async_copy(tab0.at[idxb.at[b, 0, t]],
                                              rows.at[b, t], sem).wait()

                @pl.when(cid == 1)
                def _():
                    for t in range(k_batches):
                        pltpu.make_async_copy(tab1.at[idxb.at[b, 0, t]],
                                              rows.at[b, t], sem).wait()

                for t in range(k_batches):
                    pltpu.sync_copy(rows.at[b, t], acc.at[idxb.at[b, 1, t]],
                                    add=True)

            npairs = nch // 2
            load_fire(0, 0)

            def pair(p, carry):
                load_fire(2 * p + 1, 1)
                drain_scatter(0)

                @pl.when(p < npairs - 1)
                def _():
                    load_fire(2 * p + 2, 0)

                drain_scatter(1)
                return carry

            lax.fori_loop(0, npairs, pair, 0)
            plsc.subcore_barrier()

            @pl.when(cid == 0)
            def _():
                pltpu.sync_copy(acc.at[pl.ds(base + r0, rpt)],
                                out0.at[pl.ds(r0, rpt)])

            @pl.when(cid == 1)
            def _():
                pltpu.sync_copy(acc.at[pl.ds(base + r0, rpt)],
                                out1.at[pl.ds(r0, rpt)])

            plsc.subcore_barrier()

    return k


def _multi_seg_sum(ekeys, bases, tabs, src2d, dst2d, zeros):
    jobs = []
    args = []
    for ekey, base in zip(ekeys, bases):
        e, kb, nch = _EDGE_CFG[ekey]
        n_acc = _N_ACC[ekey.split("_")[-1]]
        jobs.append((n_acc, base, 16 * 128 * kb * nch, kb, nch))
        args += [tabs[ekey][0], tabs[ekey][1], src2d[ekey], dst2d[ekey]]
    res = _multi_seg_sum_kernel(tuple(jobs))(*args, zeros)
    return {ekey: (res[2 * i], res[2 * i + 1])
            for i, ekey in enumerate(ekeys)}


@functools.lru_cache(maxsize=None)
def _seg_sum_kernel(n_acc, e_pad, k_batches, nch):
    es = e_pad // 16
    rs = es // 128
    rpt = n_acc // 16
    mesh = plsc.VectorSubcoreMesh(core_axis_name="c", subcore_axis_name="s")

    @functools.partial(
        pl.kernel,
        out_type=(jax.ShapeDtypeStruct((n_acc, 32), jnp.float32),
                  jax.ShapeDtypeStruct((n_acc, 32), jnp.float32)),
        mesh=mesh,
        scratch_types=[
            pltpu.VMEM_SHARED((n_acc, 32), jnp.float32),
            pltpu.VMEM((2, 2, k_batches, 128), jnp.int32),
            pltpu.VMEM((2, k_batches, 128, 32), jnp.float32),
            pltpu.SemaphoreType.DMA,
        ],
        compiler_params=pltpu.CompilerParams(use_tc_tiling_on_sc=False),
    )
    def k(tab0, tab1, src2d, dst2d, zeros, out0, out1, acc, idxb, rows, sem):
        cid = lax.axis_index("c")
        sid = lax.axis_index("s")
        r0 = sid * rpt
        pltpu.sync_copy(zeros.at[pl.ds(0, rpt)], acc.at[pl.ds(r0, rpt)])
        plsc.subcore_barrier()

        def load_fire(j, b):
            rb = sid * rs + j * k_batches
            pltpu.sync_copy(src2d.at[pl.ds(rb, k_batches)], idxb.at[b, 0])
            pltpu.sync_copy(dst2d.at[pl.ds(rb, k_batches)], idxb.at[b, 1])

            @pl.when(cid == 0)
            def _():
                for t in range(k_batches):
                    pltpu.async_copy(tab0.at[idxb.at[b, 0, t]],
                                     rows.at[b, t], sem)

            @pl.when(cid == 1)
            def _():
                for t in range(k_batches):
                    pltpu.async_copy(tab1.at[idxb.at[b, 0, t]],
                                     rows.at[b, t], sem)

        def drain_scatter(b):
            @pl.when(cid == 0)
            def _():
                for t in range(k_batches):
                    pltpu.make_async_copy(tab0.at[idxb.at[b, 0, t]],
                                          rows.at[b, t], sem).wait()

            @pl.when(cid == 1)
            def _():
                for t in range(k_batches):
                    pltpu.make_async_copy(tab1.at[idxb.at[b, 0, t]],
                                          rows.at[b, t], sem).wait()

            for t in range(k_batches):
                pltpu.sync_copy(rows.at[b, t], acc.at[idxb.at[b, 1, t]],
                                add=True)

        npairs = nch // 2
        load_fire(0, 0)

        def pair(p, carry):
            load_fire(2 * p + 1, 1)
            drain_scatter(0)

            @pl.when(p < npairs - 1)
            def _():
                load_fire(2 * p + 2, 0)

            drain_scatter(1)
            return carry

        lax.fori_loop(0, npairs, pair, 0)
        plsc.subcore_barrier()

        @pl.when(cid == 0)
        def _():
            pltpu.sync_copy(acc.at[pl.ds(r0, rpt)], out0.at[pl.ds(r0, rpt)])

        @pl.when(cid == 1)
        def _():
            pltpu.sync_copy(acc.at[pl.ds(r0, rpt)], out1.at[pl.ds(r0, rpt)])

    return k


def _seg_sum(ekey, tab0, tab1, src2d, dst2d, zeros):
    e, kb, nch = _EDGE_CFG[ekey]
    n_acc = _N_ACC[ekey.split("_")[-1]]
    e_pad = 16 * 128 * kb * nch
    return _seg_sum_kernel(n_acc, e_pad, kb, nch)(tab0, tab1, src2d, dst2d, zeros)


def _inv_counts(cnt_slab):
    n_acc = cnt_slab.shape[0]

    def body(c_ref, o_ref):
        o_ref[...] = 1.0 / jnp.maximum(c_ref[...][:, 0:1], 1.0)

    return pl.pallas_call(
        body,
        grid=(pl.cdiv(n_acc, 2048),),
        in_specs=[pl.BlockSpec((2048, 32), lambda i: (i, 0))],
        out_specs=pl.BlockSpec((2048, 1), lambda i: (i, 0)),
        out_shape=jax.ShapeDtypeStruct((n_acc, 1), jnp.float32),
    )(cnt_slab)


def _combine(n_rows, cnts, contribs, x_terms, bias, head, block=1024):
    grid = (pl.cdiv(n_rows, block),)
    n_cnt, n_s, n_x = len(cnts), len(contribs), len(x_terms)

    def body(*refs):
        i = 0
        cnt_refs = refs[i:i + n_cnt]; i += n_cnt
        s_refs = refs[i:i + n_s]; i += n_s
        wl_refs = refs[i:i + n_s]; i += n_s
        x_refs = refs[i:i + n_x]; i += n_x
        wx_refs = refs[i:i + n_x]; i += n_x
        b_ref = refs[i]; i += 1
        if head is not None:
            w1_ref, b1_ref, w2_ref, b2_ref = refs[i:i + 4]; i += 4
        out_refs = refs[i:]

        acc = jnp.broadcast_to(b_ref[0], (block, _HIDDEN))
        for xr, wr in zip(x_refs, wx_refs):
            acc = acc + jnp.dot(xr[...], wr[...],
                                preferred_element_type=jnp.float32)
        inv = [cr[...] for cr in cnt_refs]
        for (ci, _, _), sr, wl in zip(contribs, s_refs, wl_refs):
            acc = acc + jnp.dot(sr[...] * inv[ci], wl[...],
                                preferred_element_type=jnp.float32)
        h = jnp.maximum(acc, 0.0)
        if head is None:
            out_refs[0][...] = h[:, :32]
            out_refs[1][...] = h[:, 32:]
        else:
            u = jnp.maximum(jnp.dot(h, w1_ref[...],
                                    preferred_element_type=jnp.float32)
                            + b1_ref[0], 0.0)
            out_refs[0][...] = (jnp.dot(u, w2_ref[...],
                                        preferred_element_type=jnp.float32)
                               + b2_ref[0])

    def row_spec(w):
        return pl.BlockSpec((block, w), lambda i: (i, 0))

    def full_spec(a):
        nd = a.ndim
        return pl.BlockSpec(a.shape, lambda i, _nd=nd: (0,) * _nd)

    args = []
    specs = []
    for c in cnts:
        args.append(c); specs.append(row_spec(1))
    for _, s, _ in contribs:
        args.append(s); specs.append(row_spec(32))
    for _, _, wl in contribs:
        args.append(wl); specs.append(full_spec(wl))
    for x, _ in x_terms:
        args.append(x); specs.append(row_spec(x.shape[1]))
    for _, w in x_terms:
        args.append(w); specs.append(full_spec(w))
    b2d = bias.reshape(1, _HIDDEN)
    args.append(b2d); specs.append(full_spec(b2d))
    if head is not None:
        w1, b1, w2, b2 = head
        for a in (w1, b1.reshape(1, -1), w2, b2.reshape(1, -1)):
            args.append(a); specs.append(full_spec(a))
        od = head[2].shape[1]
        out_shape = [jax.ShapeDtypeStruct((n_rows, od), jnp.float32)]
        out_specs = [pl.BlockSpec((block, od), lambda i: (i, 0))]
    else:
        out_shape = [jax.ShapeDtypeStruct((n_rows, 32), jnp.float32)] * 2
        out_specs = [pl.BlockSpec((block, 32), lambda i: (i, 0))] * 2

    res = pl.pallas_call(
        body,
        grid=grid,
        in_specs=specs,
        out_specs=out_specs,
        out_shape=out_shape,
    )(*args)
    return res[0] if head is not None else tuple(res)


def kernel(x_bus, x_generator, x_load, x_shunt,
           edge_index_bus_ac_line_bus, edge_index_bus_transformer_bus,
           edge_index_generator_generator_link_bus,
           edge_index_bus_generator_link_generator,
           edge_index_load_load_link_bus, edge_index_bus_load_link_load,
           edge_index_shunt_shunt_link_bus, edge_index_bus_shunt_link_shunt,
           params):
    x0 = {"bus": x_bus, "generator": x_generator,
          "load": x_load, "shunt": x_shunt}
    ei = {
        "bus_ac_line_bus": edge_index_bus_ac_line_bus,
        "bus_transformer_bus": edge_index_bus_transformer_bus,
        "generator_generator_link_bus": edge_index_generator_generator_link_bus,
        "bus_generator_link_generator": edge_index_bus_generator_link_generator,
        "load_load_link_bus": edge_index_load_load_link_bus,
        "bus_load_link_load": edge_index_bus_load_link_load,
        "shunt_shunt_link_bus": edge_index_shunt_shunt_link_bus,
        "bus_shunt_link_shunt": edge_index_bus_shunt_link_shunt,
    }

    zeros = jnp.zeros((_ZROWS, 32), jnp.float32)

    src2d, dst2d = {}, {}
    for et in _EDGE_TYPES:
        ekey = _ek(et)
        e, kb, nch = _EDGE_CFG[ekey]
        e_pad = 16 * 128 * kb * nch
        s = ei[ekey][0].astype(jnp.int32)
        d = ei[ekey][1].astype(jnp.int32)
        pad = e_pad - e
        base = _SLAB_BASE.get(ekey, 0)
        s = jnp.concatenate([s, jnp.zeros((pad,), jnp.int32)])
        d = jnp.concatenate([d + base,
                             jnp.full((pad,), base + _N_NODES[et[2]],
                                      jnp.int32)])
        src2d[ekey] = s.reshape(e_pad // 128, 128)
        dst2d[ekey] = d.reshape(e_pad // 128, 128)

    tab_pad = {nt: jnp.pad(x0[nt], ((0, 0), (0, 32 - _IN_DIMS[nt])))
               for nt in _NODE_TYPES}
    tab_one = {nt: jnp.ones((_N_NODES[nt], 32), jnp.float32)
               for nt in _NODE_TYPES}

    edge_types_of = lambda layer: (
        [et for et in _EDGE_TYPES if et[2] in ("bus", "generator")]
        if layer == 2 else _EDGE_TYPES)
    dst_types_of = lambda layer: (
        ["bus", "generator"] if layer == 2 else _NODE_TYPES)

    cnt = {}
    h_half = None
    out_bus = out_gen = None

    for layer in range(3):
        if layer == 0:
            tabs = {_ek(et): (tab_pad[et[0]], tab_one[et[0]])
                    for et in _EDGE_TYPES}
        else:
            tabs = {_ek(et): h_half[et[0]] for et in _EDGE_TYPES}
        res_all = {}
        res_all.update(_multi_seg_sum(
            ["bus_ac_line_bus", "bus_transformer_bus"], [0, 0],
            tabs, src2d, dst2d, zeros))
        res_all.update(_multi_seg_sum(
            ["generator_generator_link_bus", "load_load_link_bus",
             "shunt_shunt_link_bus"], [0, 0, 0],
            tabs, src2d, dst2d, zeros))
        if layer < 2:
            res_all.update(_multi_seg_sum(
                ["bus_generator_link_generator", "bus_load_link_load",
                 "bus_shunt_link_shunt"],
                [_SLAB_BASE["bus_generator_link_generator"],
                 _SLAB_BASE["bus_load_link_load"],
                 _SLAB_BASE["bus_shunt_link_shunt"]],
                tabs, src2d, dst2d, zeros))
        else:
            res_all.update(_multi_seg_sum(
                ["bus_generator_link_generator"],
                [_SLAB_BASE["bus_generator_link_generator"]],
                tabs, src2d, dst2d, zeros))

        segsums = {}
        for et in edge_types_of(layer):
            ekey = _ek(et)
            res = res_all[ekey]
            if layer == 0:
                cnt[ekey] = _inv_counts(res[1])
                wl = params["l0_%s_Wl" % ekey]
                wl = jnp.pad(wl, ((0, 32 - wl.shape[0]), (0, 0)))
                segsums[ekey] = ([res[0]], [wl])
            else:
                wl = params["l%d_%s_Wl" % (layer, ekey)]
                segsums[ekey] = ([res[0], res[1]], [wl[:32], wl[32:]])

        new_h = {}
        for nt in dst_types_of(layer):
            in_ets = [et for et in edge_types_of(layer) if et[2] == nt]
            cnts, contribs = [], []
            bias = 0.0
            x_terms = []
            for et in in_ets:
                ekey = _ek(et)
                ci = len(cnts)
                cnts.append(cnt[ekey])
                s_arrays, wl_parts = segsums[ekey]
                for s, wl in zip(s_arrays, wl_parts):
                    contribs.append((ci, s, wl))
                wr = params["l%d_%s_Wr" % (layer, ekey)]
                if layer == 0:
                    x_terms.append((x0[nt], wr))
                else:
                    lo, hi = h_half[nt]
                    x_terms.append((lo, wr[:32]))
                    x_terms.append((hi, wr[32:]))
                bias = bias + params["l%d_%s_bl" % (layer, ekey)]
            head = None
            if layer == 2:
                head = (params[nt + "_W1"], params[nt + "_b1"],
                        params[nt + "_W2"], params[nt + "_b2"])
            res = _combine(_N_NODES[nt], cnts, contribs, x_terms, bias, head)
            if layer == 2:
                if nt == "bus":
                    out_bus = res
                else:
                    out_gen = res
            else:
                new_h[nt] = res
        if layer < 2:
            h_half = new_h

    return out_bus, out_gen


# speedup vs baseline: 1.0714x; 1.0015x over previous
import functools

import jax
import jax.numpy as jnp
from jax import lax
from jax.experimental import pallas as pl
from jax.experimental.pallas import tpu as pltpu
from jax.experimental.pallas import tpu_sc as plsc

_EDGE_TYPES = [
    ("bus", "ac_line", "bus"),
    ("bus", "transformer", "bus"),
    ("generator", "generator_link", "bus"),
    ("bus", "generator_link", "generator"),
    ("load", "load_link", "bus"),
    ("bus", "load_link", "load"),
    ("shunt", "shunt_link", "bus"),
    ("bus", "shunt_link", "shunt"),
]
_N_NODES = {"bus": 50000, "generator": 10000, "load": 25000, "shunt": 5000}
_IN_DIMS = {"bus": 32, "generator": 16, "load": 16, "shunt": 8}
_HIDDEN = 64
_NODE_TYPES = ["bus", "generator", "load", "shunt"]

_EDGE_CFG = {
    "bus_ac_line_bus": (800000, 2, 196),
    "bus_transformer_bus": (100000, 2, 26),
    "generator_generator_link_bus": (10000, 1, 6),
    "bus_generator_link_generator": (10000, 1, 6),
    "load_load_link_bus": (25000, 1, 14),
    "bus_load_link_load": (25000, 1, 14),
    "shunt_shunt_link_bus": (5000, 1, 4),
    "bus_shunt_link_shunt": (5000, 1, 4),
}
_N_ACC = {"bus": 50048, "generator": 10112, "load": 25088, "shunt": 5120}
_ZROWS = _N_ACC["bus"] // 16
_SLAB_BASE = {"bus_generator_link_generator": 0,
              "bus_load_link_load": 10112,
              "bus_shunt_link_shunt": 35200}


def _ek(et):
    return et[0] + "_" + et[1] + "_" + et[2]


@functools.lru_cache(maxsize=None)
def _multi_seg_sum_kernel(jobs):
    kmax = max(j[3] for j in jobs)
    slab = max(j[0] + j[1] for j in jobs)
    mesh = plsc.VectorSubcoreMesh(core_axis_name="c", subcore_axis_name="s")
    out_type = []
    for n_acc, _, _, _, _ in jobs:
        out_type.append(jax.ShapeDtypeStruct((n_acc, 32), jnp.float32))
        out_type.append(jax.ShapeDtypeStruct((n_acc, 32), jnp.float32))

    @functools.partial(
        pl.kernel,
        out_type=tuple(out_type),
        mesh=mesh,
        scratch_types=[
            pltpu.VMEM_SHARED((slab, 32), jnp.float32),
            pltpu.VMEM((2, 2, kmax, 128), jnp.int32),
            pltpu.VMEM((2, kmax, 128, 32), jnp.float32),
            pltpu.SemaphoreType.DMA,
        ],
        compiler_params=pltpu.CompilerParams(use_tc_tiling_on_sc=False),
    )
    def k(*refs):
        n = len(jobs)
        ins = refs[:4 * n + 1]
        outs = refs[4 * n + 1:4 * n + 1 + 2 * n]
        acc, idxb, rows, sem = refs[4 * n + 1 + 2 * n:]
        zeros = ins[4 * n]
        cid = lax.axis_index("c")
        sid = lax.axis_index("s")

        for ji, (n_acc, base, e_pad, k_batches, nch) in enumerate(jobs):
            tab0, tab1, src2d, dst2d = ins[4 * ji:4 * ji + 4]
            out0, out1 = outs[2 * ji], outs[2 * ji + 1]
            rs = (e_pad // 16) // 128
            rpt = n_acc // 16
            r0 = sid * rpt
            pltpu.sync_copy(zeros.at[pl.ds(0, rpt)],
                            acc.at[pl.ds(base + r0, rpt)])
            plsc.subcore_barrier()

            def load_fire(j, b):
                rb = sid * rs + j * k_batches
                pltpu.sync_copy(src2d.at[pl.ds(rb, k_batches)], idxb.at[b, 0])
                pltpu.sync_copy(dst2d.at[pl.ds(rb, k_batches)], idxb.at[b, 1])

                @pl.when(cid == 0)
                def _():
                    for t in range(k_batches):
                        pltpu.async_copy(tab0.at[idxb.at[b, 0, t]],
                                         rows.at[b, t], sem)

                @pl.when(cid == 1)
                def _():
                    for t in range(k_batches):
                        pltpu.async_copy(tab1.at[idxb.at[b, 0, t]],
                                         rows.at[b, t], sem)

            def drain_scatter(b):
                @pl.when(cid == 0)
                def _():
                    for t in range(k_batches):
                        pltpu.make_async_copy(tab0.at[idxb.at[b, 0, t]],
                                              rows.at[b, t], sem).wait()

                @pl.when(cid == 1)
                def _():
                    for t in range(k_batches):
                        pltpu.make_async_copy(tab1.at[idxb.at[b, 0, t]],
                                              rows.at[b, t], sem).wait()

                for t in range(k_batches):
                    pltpu.sync_copy(rows.at[b, t], acc.at[idxb.at[b, 1, t]],
                                    add=True)

            npairs = nch // 2
            load_fire(0, 0)

            def pair(p, carry):
                load_fire(2 * p + 1, 1)
                drain_scatter(0)

                @pl.when(p < npairs - 1)
                def _():
                    load_fire(2 * p + 2, 0)

                drain_scatter(1)
                return carry

            lax.fori_loop(0, npairs, pair, 0)
            plsc.subcore_barrier()

            @pl.when(cid == 0)
            def _():
                pltpu.sync_copy(acc.at[pl.ds(base + r0, rpt)],
                                out0.at[pl.ds(r0, rpt)])

            @pl.when(cid == 1)
            def _():
                pltpu.sync_copy(acc.at[pl.ds(base + r0, rpt)],
                                out1.at[pl.ds(r0, rpt)])

            plsc.subcore_barrier()

    return k


def _multi_seg_sum(ekeys, bases, tabs, src2d, dst2d, zeros):
    jobs = []
    args = []
    for ekey, base in zip(ekeys, bases):
        e, kb, nch = _EDGE_CFG[ekey]
        n_acc = _N_ACC[ekey.split("_")[-1]]
        jobs.append((n_acc, base, 16 * 128 * kb * nch, kb, nch))
        args += [tabs[ekey][0], tabs[ekey][1], src2d[ekey], dst2d[ekey]]
    res = _multi_seg_sum_kernel(tuple(jobs))(*args, zeros)
    return {ekey: (res[2 * i], res[2 * i + 1])
            for i, ekey in enumerate(ekeys)}


def _inv_counts(cnt_slab):
    n_acc = cnt_slab.shape[0]

    def body(c_ref, o_ref):
        o_ref[...] = 1.0 / jnp.maximum(c_ref[...][:, 0:1], 1.0)

    return pl.pallas_call(
        body,
        grid=(pl.cdiv(n_acc, 2048),),
        in_specs=[pl.BlockSpec((2048, 32), lambda i: (i, 0))],
        out_specs=pl.BlockSpec((2048, 1), lambda i: (i, 0)),
        out_shape=jax.ShapeDtypeStruct((n_acc, 1), jnp.float32),
    )(cnt_slab)


def _combine(n_rows, cnts, contribs, x_terms, bias, head, block=1024):
    grid = (pl.cdiv(n_rows, block),)
    n_cnt, n_s, n_x = len(cnts), len(contribs), len(x_terms)

    def body(*refs):
        i = 0
        cnt_refs = refs[i:i + n_cnt]; i += n_cnt
        s_refs = refs[i:i + n_s]; i += n_s
        wl_refs = refs[i:i + n_s]; i += n_s
        x_refs = refs[i:i + n_x]; i += n_x
        wx_refs = refs[i:i + n_x]; i += n_x
        b_ref = refs[i]; i += 1
        if head is not None:
            w1_ref, b1_ref, w2_ref, b2_ref = refs[i:i + 4]; i += 4
        out_refs = refs[i:]

        acc = jnp.broadcast_to(b_ref[0], (block, _HIDDEN))
        for xr, wr in zip(x_refs, wx_refs):
            acc = acc + jnp.dot(xr[...], wr[...],
                                preferred_element_type=jnp.float32)
        inv = [cr[...] for cr in cnt_refs]
        for (ci, _, _), sr, wl in zip(contribs, s_refs, wl_refs):
            acc = acc + jnp.dot(sr[...] * inv[ci], wl[...],
                                preferred_element_type=jnp.float32)
        h = jnp.maximum(acc, 0.0)
        if head is None:
            out_refs[0][...] = h[:, :32]
            out_refs[1][...] = h[:, 32:]
        else:
            u = jnp.maximum(jnp.dot(h, w1_ref[...],
                                    preferred_element_type=jnp.float32)
                            + b1_ref[0], 0.0)
            out_refs[0][...] = (jnp.dot(u, w2_ref[...],
                                        preferred_element_type=jnp.float32)
                               + b2_ref[0])

    def row_spec(w):
        return pl.BlockSpec((block, w), lambda i: (i, 0))

    def full_spec(a):
        nd = a.ndim
        return pl.BlockSpec(a.shape, lambda i, _nd=nd: (0,) * _nd)

    args = []
    specs = []
    for c in cnts:
        args.append(c); specs.append(row_spec(1))
    for _, s, _ in contribs:
        args.append(s); specs.append(row_spec(32))
    for _, _, wl in contribs:
        args.append(wl); specs.append(full_spec(wl))
    for x, _ in x_terms:
        args.append(x); specs.append(row_spec(x.shape[1]))
    for _, w in x_terms:
        args.append(w); specs.append(full_spec(w))
    b2d = bias.reshape(1, _HIDDEN)
    args.append(b2d); specs.append(full_spec(b2d))
    if head is not None:
        w1, b1, w2, b2 = head
        for a in (w1, b1.reshape(1, -1), w2, b2.reshape(1, -1)):
            args.append(a); specs.append(full_spec(a))
        od = head[2].shape[1]
        out_shape = [jax.ShapeDtypeStruct((n_rows, od), jnp.float32)]
        out_specs = [pl.BlockSpec((block, od), lambda i: (i, 0))]
    else:
        out_shape = [jax.ShapeDtypeStruct((n_rows, 32), jnp.float32)] * 2
        out_specs = [pl.BlockSpec((block, 32), lambda i: (i, 0))] * 2

    res = pl.pallas_call(
        body,
        grid=grid,
        in_specs=specs,
        out_specs=out_specs,
        out_shape=out_shape,
    )(*args)
    return res[0] if head is not None else tuple(res)


def kernel(x_bus, x_generator, x_load, x_shunt,
           edge_index_bus_ac_line_bus, edge_index_bus_transformer_bus,
           edge_index_generator_generator_link_bus,
           edge_index_bus_generator_link_generator,
           edge_index_load_load_link_bus, edge_index_bus_load_link_load,
           edge_index_shunt_shunt_link_bus, edge_index_bus_shunt_link_shunt,
           params):
    x0 = {"bus": x_bus, "generator": x_generator,
          "load": x_load, "shunt": x_shunt}
    ei = {
        "bus_ac_line_bus": edge_index_bus_ac_line_bus,
        "bus_transformer_bus": edge_index_bus_transformer_bus,
        "generator_generator_link_bus": edge_index_generator_generator_link_bus,
        "bus_generator_link_generator": edge_index_bus_generator_link_generator,
        "load_load_link_bus": edge_index_load_load_link_bus,
        "bus_load_link_load": edge_index_bus_load_link_load,
        "shunt_shunt_link_bus": edge_index_shunt_shunt_link_bus,
        "bus_shunt_link_shunt": edge_index_bus_shunt_link_shunt,
    }

    zeros = jnp.zeros((_ZROWS, 32), jnp.float32)

    src2d, dst2d = {}, {}
    for et in _EDGE_TYPES:
        ekey = _ek(et)
        e, kb, nch = _EDGE_CFG[ekey]
        e_pad = 16 * 128 * kb * nch
        s = ei[ekey][0].astype(jnp.int32)
        d = ei[ekey][1].astype(jnp.int32)
        pad = e_pad - e
        base = _SLAB_BASE.get(ekey, 0)
        s = jnp.concatenate([s, jnp.zeros((pad,), jnp.int32)])
        d = jnp.concatenate([d + base,
                             jnp.full((pad,), base + _N_NODES[et[2]],
                                      jnp.int32)])
        src2d[ekey] = s.reshape(e_pad // 128, 128)
        dst2d[ekey] = d.reshape(e_pad // 128, 128)

    tab_pad = {nt: jnp.pad(x0[nt], ((0, 0), (0, 32 - _IN_DIMS[nt])))
               for nt in _NODE_TYPES}
    tab_one = {nt: jnp.ones((_N_NODES[nt], 32), jnp.float32)
               for nt in _NODE_TYPES}

    edge_types_of = lambda layer: (
        [et for et in _EDGE_TYPES if et[2] in ("bus", "generator")]
        if layer == 2 else _EDGE_TYPES)
    dst_types_of = lambda layer: (
        ["bus", "generator"] if layer == 2 else _NODE_TYPES)

    cnt = {}
    h_half = None
    out_bus = out_gen = None

    for layer in range(3):
        if layer == 0:
            tabs = {_ek(et): (tab_pad[et[0]], tab_one[et[0]])
                    for et in _EDGE_TYPES}
        else:
            tabs = {_ek(et): h_half[et[0]] for et in _EDGE_TYPES}
        res_all = {}
        res_all.update(_multi_seg_sum(
            ["bus_ac_line_bus", "bus_transformer_bus"], [0, 0],
            tabs, src2d, dst2d, zeros))
        res_all.update(_multi_seg_sum(
            ["generator_generator_link_bus", "load_load_link_bus",
             "shunt_shunt_link_bus"], [0, 0, 0],
            tabs, src2d, dst2d, zeros))
        if layer < 2:
            res_all.update(_multi_seg_sum(
                ["bus_generator_link_generator", "bus_load_link_load",
                 "bus_shunt_link_shunt"],
                [_SLAB_BASE["bus_generator_link_generator"],
                 _SLAB_BASE["bus_load_link_load"],
                 _SLAB_BASE["bus_shunt_link_shunt"]],
                tabs, src2d, dst2d, zeros))
        else:
            res_all.update(_multi_seg_sum(
                ["bus_generator_link_generator"],
                [_SLAB_BASE["bus_generator_link_generator"]],
                tabs, src2d, dst2d, zeros))

        segsums = {}
        for et in edge_types_of(layer):
            ekey = _ek(et)
            res = res_all[ekey]
            if layer == 0:
                cnt[ekey] = _inv_counts(res[1])
                wl = params["l0_%s_Wl" % ekey]
                wl = jnp.pad(wl, ((0, 32 - wl.shape[0]), (0, 0)))
                segsums[ekey] = ([res[0]], [wl])
            else:
                wl = params["l%d_%s_Wl" % (layer, ekey)]
                segsums[ekey] = ([res[0], res[1]], [wl[:32], wl[32:]])

        new_h = {}
        for nt in dst_types_of(layer):
            in_ets = [et for et in edge_types_of(layer) if et[2] == nt]
            cnts, contribs = [], []
            bias = 0.0
            x_terms = []
            for et in in_ets:
                ekey = _ek(et)
                ci = len(cnts)
                cnts.append(cnt[ekey])
                s_arrays, wl_parts = segsums[ekey]
                for s, wl in zip(s_arrays, wl_parts):
                    contribs.append((ci, s, wl))
                wr = params["l%d_%s_Wr" % (layer, ekey)]
                if layer == 0:
                    x_terms.append((x0[nt], wr))
                else:
                    lo, hi = h_half[nt]
                    x_terms.append((lo, wr[:32]))
                    x_terms.append((hi, wr[32:]))
                bias = bias + params["l%d_%s_bl" % (layer, ekey)]
            head = None
            if layer == 2:
                head = (params[nt + "_W1"], params[nt + "_b1"],
                        params[nt + "_W2"], params[nt + "_b2"])
            res = _combine(_N_NODES[nt], cnts, contribs, x_terms, bias, head)
            if layer == 2:
                if nt == "bus":
                    out_bus = res
                else:
                    out_gen = res
            else:
                new_h[nt] = res
        if layer < 2:
            h_half = new_h

    return out_bus, out_gen
